# baseline (speedup 1.0000x reference)
import functools

import jax
import jax.numpy as jnp
from jax import lax
from jax.experimental import pallas as pl
from jax.experimental.pallas import tpu as pltpu
from jax.experimental.pallas import tpu_sc as plsc

N = 10000
E = 160000
IN = 100
OUT = 200
STEPS = 6
HID = 400
G = 64

D = 256
RANGE = 3000
NROWS_SC = 3008
DUMMY_ROW = 3000
RPT = 184
RPT_XTRA = NROWS_SC - 16 * RPT
K = 128
REG = 88
TOT_CH = 64 * REG
NBUF = 2
DUMMY_PK = DUMMY_ROW << 14

BLK = 1000
GRID = N // BLK

f32 = jnp.float32
i32 = jnp.int32


def _pad2(w, r, c):
    return jnp.pad(w, ((0, r - w.shape[0]), (0, c - w.shape[1])))


def _sc_scatter_body(m_hbm, packed_hbm, counts_hbm, zeros_hbm,
                     plo_hbm, phi_hbm, pk_tmp, src_ring, dst_ring, rows_v,
                     split_v, cnt_v, agg_lo, agg_hi, sem0, sem1, *, call):
    c = lax.axis_index("c")
    s = lax.axis_index("s")
    sems = (sem0, sem1)
    w = c * 16 + s
    row0 = s * RPT
    for agg in (agg_lo, agg_hi):
        pltpu.sync_copy(zeros_hbm.at[pl.ds(0, RPT)],
                        agg.at[pl.ds(row0, RPT)])

        @pl.when(s == 15)
        def _(agg=agg):
            pltpu.sync_copy(zeros_hbm.at[pl.ds(0, RPT_XTRA)],
                            agg.at[pl.ds(16 * RPT, RPT_XTRA)])

    pltpu.sync_copy(counts_hbm.at[w], cnt_v)
    tpairs = cnt_v[0, :][0]
    side = 2 * c + call
    start = (side * 16 + s) * REG
    plsc.subcore_barrier()

    def unpack(chunk, b):
        pltpu.sync_copy(packed_hbm.at[start + chunk], pk_tmp)
        for v in range(K // 16):
            p = pk_tmp[pl.ds(v * 16, 16)]
            src_ring[b, pl.ds(v * 16, 16)] = p & 16383
            dst_ring[b, pl.ds(v * 16, 16)] = lax.shift_right_logical(p, 14)

    def fire(b):
        pltpu.async_copy(m_hbm.at[src_ring.at[b]], rows_v.at[b], sems[b])

    for b in range(NBUF):
        unpack(b, b)
        fire(b)

    @pl.loop(0, tpairs)
    def _(t):
        for b in range(NBUF):
            pltpu.make_async_copy(m_hbm.at[src_ring.at[b]],
                                  rows_v.at[b], sems[b]).wait()
            for half, agg in ((0, agg_lo), (1, agg_hi)):
                @pl.loop(0, K)
                def _(r, b=b, half=half):
                    for v in range(8):
                        split_v[r, pl.ds(v * 16, 16)] = (
                            rows_v[b, r, pl.ds(half * 128 + v * 16, 16)])

                pltpu.sync_copy(split_v, agg.at[dst_ring.at[b]], add=True)
            unpack(NBUF * t + b + NBUF, b)
            fire(b)

    for b in range(NBUF):
        pltpu.make_async_copy(m_hbm.at[src_ring.at[b]],
                              rows_v.at[b], sems[b]).wait()

    plsc.subcore_barrier()
    for agg, out in ((agg_lo, plo_hbm), (agg_hi, phi_hbm)):
        pltpu.sync_copy(agg.at[pl.ds(row0, RPT)],
                        out.at[c].at[pl.ds(row0, RPT)])

        @pl.when(s == 15)
        def _(agg=agg, out=out):
            pltpu.sync_copy(agg.at[pl.ds(16 * RPT, RPT_XTRA)],
                            out.at[c].at[pl.ds(16 * RPT, RPT_XTRA)])


def _sc_scatter(m, packed2, counts3, zeros_rpt, call):
    return pl.kernel(
        functools.partial(_sc_scatter_body, call=call),
        out_type=(jax.ShapeDtypeStruct((2, NROWS_SC, 128), f32),
                  jax.ShapeDtypeStruct((2, NROWS_SC, 128), f32)),
        mesh=plsc.VectorSubcoreMesh(core_axis_name="c",
                                    subcore_axis_name="s"),
        scratch_types=[
            pltpu.VMEM((K,), i32),
            pltpu.VMEM((NBUF, K), i32),
            pltpu.VMEM((NBUF, K), i32),
            pltpu.VMEM((NBUF, K, D), f32),
            pltpu.VMEM((K, 128), f32),
            pltpu.VMEM((1, 16), i32),
            pltpu.VMEM_SHARED((NROWS_SC, 128), f32),
            pltpu.VMEM_SHARED((NROWS_SC, 128), f32),
            pltpu.SemaphoreType.DMA,
            pltpu.SemaphoreType.DMA,
        ],
    )(m, packed2, counts3, zeros_rpt)


def _mm_body(x_ref, w_ref, m_ref):
    m_ref[...] = jnp.dot(x_ref[...], w_ref[...], preferred_element_type=f32)


def _mm(x, w):
    return pl.pallas_call(
        _mm_body,
        grid=(GRID,),
        in_specs=[pl.BlockSpec((BLK, D), lambda i: (i, 0)),
                  pl.BlockSpec((D, D), lambda i: (0, 0))],
        out_specs=pl.BlockSpec((BLK, D), lambda i: (i, 0)),
        out_shape=jax.ShapeDtypeStruct((N, D), f32),
    )(x, w)


def _gru_compute(pal_ref, pah_ref, pbl_ref, pbh_ref, h_ref, w_ref, b_ref):
    i = pl.program_id(0)
    use_a = (i // 3) % 2 == 0
    agg_a = jnp.concatenate([pal_ref[0], pah_ref[0]], axis=1)
    agg_b = jnp.concatenate([pbl_ref[0], pbh_ref[0]], axis=1)
    agg = jnp.where(use_a, agg_a, agg_b)
    h = h_ref[...]
    dot = functools.partial(jnp.dot, preferred_element_type=f32)
    r = jax.nn.sigmoid(dot(agg, w_ref[0]) + dot(h, w_ref[3]) + b_ref[0:1, :])
    z = jax.nn.sigmoid(dot(agg, w_ref[1]) + dot(h, w_ref[4]) + b_ref[1:2, :])
    hn = dot(h, w_ref[5]) + b_ref[3:4, :]
    n = jnp.tanh(dot(agg, w_ref[2]) + b_ref[2:3, :] + r * hn)
    return (1.0 - z) * n + z * h


def _gru_body_m(pal, pah, pbl, pbh, h_ref, w_ref, b_ref, h_out, m_out):
    hnew = _gru_compute(pal, pah, pbl, pbh, h_ref, w_ref, b_ref)
    h_out[...] = hnew
    m_out[...] = jnp.dot(hnew, w_ref[6], preferred_element_type=f32)


def _gru_body_last(pal, pah, pbl, pbh, h_ref, w_ref, b_ref, h_out):
    h_out[...] = _gru_compute(pal, pah, pbl, pbh, h_ref, w_ref, b_ref)


def _gru_step(pa, pb, h, ws, b, emit_m):
    nw = ws.shape[0]
    if emit_m:
        out_shape = [jax.ShapeDtypeStruct((N, D), f32),
                     jax.ShapeDtypeStruct((N, D), f32)]
        out_specs = [pl.BlockSpec((BLK, D), lambda i: (i, 0)),
                     pl.BlockSpec((BLK, D), lambda i: (i, 0))]
        body = _gru_body_m
    else:
        out_shape = [jax.ShapeDtypeStruct((N, D), f32)]
        out_specs = [pl.BlockSpec((BLK, D), lambda i: (i, 0))]
        body = _gru_body_last
    pmap = lambda i: ((i // 3) // 2, i % 3, 0)
    return pl.pallas_call(
        body,
        grid=(GRID,),
        in_specs=[pl.BlockSpec((1, BLK, 128), pmap),
                  pl.BlockSpec((1, BLK, 128), pmap),
                  pl.BlockSpec((1, BLK, 128), pmap),
                  pl.BlockSpec((1, BLK, 128), pmap),
                  pl.BlockSpec((BLK, D), lambda i: (i, 0)),
                  pl.BlockSpec((nw, D, D), lambda i: (0, 0, 0)),
                  pl.BlockSpec((8, D), lambda i: (0, 0))],
        out_specs=out_specs,
        out_shape=out_shape,
    )(pa[0], pa[1], pb[0], pb[1], h, ws, b)


def _tail_body(h_ref, batch_ref, l1w_ref, l1b_ref, f1w_ref, f1b_ref,
               f2w_ref, f2b_ref, clsw_ref, clsb_ref, y_ref, acc):
    i = pl.program_id(0)

    @pl.when(i == 0)
    def _():
        acc[...] = jnp.zeros_like(acc)

    out = jax.nn.relu(h_ref[...])
    b = batch_ref[0, 0, :]
    seg = lax.broadcasted_iota(i32, (G, BLK), 0)
    onehot = jnp.where(seg == b[None, :], 1.0, 0.0).astype(f32)
    acc[...] += jnp.dot(onehot, out, preferred_element_type=f32)

    @pl.when(i == GRID - 1)
    def _():
        dot = functools.partial(jnp.dot, preferred_element_type=f32)
        pooled = acc[...]
        a = jax.nn.relu(dot(pooled, l1w_ref[...]) + l1b_ref[0:1, :])
        a = jax.nn.relu(dot(a, f1w_ref[...]) + f1b_ref[0:1, :])
        a = jax.nn.relu(dot(a, f2w_ref[...]) + f2b_ref[0:1, :])
        logits = dot(a, clsw_ref[...]) + clsb_ref[0:1, :]
        lane = lax.broadcasted_iota(i32, (G, 8), 1)
        logits = jnp.where(lane < 2, logits, -1e30)
        mx = jnp.max(logits, axis=1, keepdims=True)
        e = jnp.exp(logits - mx)
        y_ref[...] = e / jnp.sum(e, axis=1, keepdims=True)


def _tail(h, batch3, l1w, l1b, f1w, f1b, f2w, f2b, clsw, clsb):
    return pl.pallas_call(
        _tail_body,
        grid=(GRID,),
        in_specs=[pl.BlockSpec((BLK, D), lambda i: (i, 0)),
                  pl.BlockSpec((1, 1, BLK), lambda i: (i, 0, 0)),
                  pl.BlockSpec((D, HID), lambda i: (0, 0)),
                  pl.BlockSpec((1, HID), lambda i: (0, 0)),
                  pl.BlockSpec((HID, D), lambda i: (0, 0)),
                  pl.BlockSpec((1, D), lambda i: (0, 0)),
                  pl.BlockSpec((D, HID), lambda i: (0, 0)),
                  pl.BlockSpec((1, HID), lambda i: (0, 0)),
                  pl.BlockSpec((HID, 8), lambda i: (0, 0)),
                  pl.BlockSpec((1, 8), lambda i: (0, 0))],
        out_specs=pl.BlockSpec((G, 8), lambda i: (0, 0)),
        out_shape=jax.ShapeDtypeStruct((G, 8), f32),
        scratch_shapes=[pltpu.VMEM((G, D), f32)],
    )(h, batch3, l1w, l1b, f1w, f1b, f2w, f2b, clsw, clsb)


def _ceil8(x):
    return (x + 7) // 8 * 8


def _partition_edges(edge_index):
    src = edge_index[0]
    dst = edge_index[1]
    sid = dst // RANGE
    local = dst - sid * RANGE
    packed = (local << 14) | src

    onehot = sid[None, :] == jnp.arange(4, dtype=i32)[:, None]
    cums = jnp.cumsum(onehot.astype(i32), axis=1)
    n_edges = cums[:, -1]
    quota = jnp.maximum((n_edges + 15) // 16, 1)
    rank = jnp.take_along_axis(cums, sid[None, :], axis=0)[0] - 1
    worker = rank // quota[sid]
    within = rank - worker * quota[sid]
    pos = (sid * 16 + worker) * (REG * K) + within
    buf = jnp.full((TOT_CH * K,), DUMMY_PK, i32).at[pos].set(packed)
    packed2 = buf.reshape(TOT_CH, K)

    wrange = jnp.arange(32, dtype=i32)
    svec = wrange % 16
    tables = []
    for a in range(2):
        rvec = (wrange // 16) * 2 + a
        e_ws = jnp.clip(n_edges[rvec] - svec * quota[rvec], 0, quota[rvec])
        trips = (e_ws + K - 1) // K
        tpairs = (trips + 1) // 2
        tables.append(
            jnp.broadcast_to(tpairs.reshape(32, 1, 1), (32, 1, 16))
            .astype(i32))
    return packed2, tables


def kernel(x, edge_index, batch, ggnn_weight, W_ih, W_hh, b_ih, b_hh,
           l1_w, l1_b, f1_w, f1_b, f2_w, f2_b, cls_w, cls_b):
    h0 = jnp.pad(x, ((0, 0), (0, D - IN))).astype(f32)
    packed2, tables = _partition_edges(edge_index)
    zeros_rpt = jnp.zeros((RPT, 128), f32)
    batch3 = batch.reshape(GRID, 1, BLK)

    wg = [_pad2(ggnn_weight[i], D, D) for i in range(STEPS)]
    wir = _pad2(W_ih[0:OUT].T, D, D)
    wiz = _pad2(W_ih[OUT:2 * OUT].T, D, D)
    win = _pad2(W_ih[2 * OUT:].T, D, D)
    whr = _pad2(W_hh[0:OUT].T, D, D)
    whz = _pad2(W_hh[OUT:2 * OUT].T, D, D)
    whn = _pad2(W_hh[2 * OUT:].T, D, D)
    br = jnp.pad(b_ih[0:OUT] + b_hh[0:OUT], (0, D - OUT))
    bz = jnp.pad(b_ih[OUT:2 * OUT] + b_hh[OUT:2 * OUT], (0, D - OUT))
    bin_ = jnp.pad(b_ih[2 * OUT:], (0, D - OUT))
    bhn = jnp.pad(b_hh[2 * OUT:], (0, D - OUT))
    bmat = jnp.stack([br, bz, bin_, bhn] + [jnp.zeros((D,), f32)] * 4)

    l1wt = _pad2(l1_w.T, D, HID)
    f1wt = _pad2(f1_w.T, HID, D)
    f2wt = _pad2(f2_w.T, D, HID)
    clswt = _pad2(cls_w.T, HID, 8)
    l1b2 = l1_b.reshape(1, HID)
    f1b2 = _pad2(f1_b.reshape(1, OUT), 1, D)
    f2b2 = f2_b.reshape(1, HID)
    clsb2 = _pad2(cls_b.reshape(1, 2), 1, 8)

    h = h0
    m = _mm(h, wg[0])
    for i in range(STEPS):
        pa = _sc_scatter(m, packed2, tables[0], zeros_rpt, 0)
        pb = _sc_scatter(m, packed2, tables[1], zeros_rpt, 1)
        if i < STEPS - 1:
            ws = jnp.stack([wir, wiz, win, whr, whz, whn, wg[i + 1]])
            h, m = _gru_step(pa, pb, h, ws, bmat, True)
        else:
            ws = jnp.stack([wir, wiz, win, whr, whz, whn])
            (h,) = _gru_step(pa, pb, h, ws, bmat, False)

    y8 = _tail(h, batch3, l1wt, l1b2, f1wt, f1b2, f2wt, f2b2, clswt, clsb2)
    return y8[:, :2]


# speedup vs baseline: 3.4448x; 3.4448x over previous
import functools

import jax
import jax.numpy as jnp
from jax import lax
from jax.experimental import pallas as pl
from jax.experimental.pallas import tpu as pltpu
from jax.experimental.pallas import tpu_sc as plsc

N = 10000
E = 160000
IN = 100
OUT = 200
STEPS = 6
HID = 400
G = 64

D = 256
DH = 128
NROWS_SC = 10016
DUMMY_DST = 10008
RPT = 624
RPT_XTRA = NROWS_SC - 16 * RPT
K = 128
CHUNKS = 80
NBUF = 2
E_PAD = 16 * CHUNKS * K

BLK = 1000
GRID = N // BLK

f32 = jnp.float32


def _pad2(w, r, c):
    return jnp.pad(w, ((0, r - w.shape[0]), (0, c - w.shape[1])))


def _sc_scatter_body(mlo_hbm, mhi_hbm, packed_hbm, zeros_hbm,
                     plo_hbm, phi_hbm, pk_all, src_ring, dst_ring, rows_v,
                     agg_s, sem0, sem1):
    c = lax.axis_index("c")
    s = lax.axis_index("s")
    sems = (sem0, sem1)
    row0 = s * RPT
    pltpu.sync_copy(zeros_hbm.at[pl.ds(0, RPT)], agg_s.at[pl.ds(row0, RPT)])

    @pl.when(s == 15)
    def _():
        pltpu.sync_copy(zeros_hbm.at[pl.ds(0, RPT_XTRA)],
                        agg_s.at[pl.ds(16 * RPT, RPT_XTRA)])

    pltpu.sync_copy(packed_hbm.at[pl.ds(s * CHUNKS, CHUNKS)], pk_all)
    plsc.subcore_barrier()

    def unpack(chunk, b):
        for v in range(K // 16):
            p = pk_all[chunk, pl.ds(v * 16, 16)]
            src_ring[b, pl.ds(v * 16, 16)] = p & 16383
            dst_ring[b, pl.ds(v * 16, 16)] = lax.shift_right_logical(p, 14)

    def run(m_hbm):
        def fire(b):
            pltpu.async_copy(m_hbm.at[src_ring.at[b]], rows_v.at[b],
                             sems[b])

        for b in range(NBUF):
            unpack(b, b)
            fire(b)

        @pl.loop(0, CHUNKS, step=NBUF)
        def _(j0):
            for b in range(NBUF):
                j = j0 + b
                pltpu.make_async_copy(m_hbm.at[src_ring.at[b]],
                                      rows_v.at[b], sems[b]).wait()
                pltpu.sync_copy(rows_v.at[b], agg_s.at[dst_ring.at[b]],
                                add=True)

                @pl.when(j + NBUF < CHUNKS)
                def _():
                    unpack(j + NBUF, b)
                    fire(b)

    @pl.when(c == 0)
    def _():
        run(mlo_hbm)

    @pl.when(c == 1)
    def _():
        run(mhi_hbm)

    plsc.subcore_barrier()

    @pl.when(c == 0)
    def _():
        pltpu.sync_copy(agg_s.at[pl.ds(row0, RPT)],
                        plo_hbm.at[pl.ds(row0, RPT)])

        @pl.when(s == 15)
        def _():
            pltpu.sync_copy(agg_s.at[pl.ds(16 * RPT, RPT_XTRA)],
                            plo_hbm.at[pl.ds(16 * RPT, RPT_XTRA)])

    @pl.when(c == 1)
    def _():
        pltpu.sync_copy(agg_s.at[pl.ds(row0, RPT)],
                        phi_hbm.at[pl.ds(row0, RPT)])

        @pl.when(s == 15)
        def _():
            pltpu.sync_copy(agg_s.at[pl.ds(16 * RPT, RPT_XTRA)],
                            phi_hbm.at[pl.ds(16 * RPT, RPT_XTRA)])


def _sc_scatter(mlo, mhi, packed2, zeros_rpt):
    return pl.kernel(
        _sc_scatter_body,
        out_type=(jax.ShapeDtypeStruct((NROWS_SC, DH), f32),
                  jax.ShapeDtypeStruct((NROWS_SC, DH), f32)),
        mesh=plsc.VectorSubcoreMesh(core_axis_name="c",
                                    subcore_axis_name="s"),
        scratch_types=[
            pltpu.VMEM((CHUNKS, K), jnp.int32),
            pltpu.VMEM((NBUF, K), jnp.int32),
            pltpu.VMEM((NBUF, K), jnp.int32),
            pltpu.VMEM((NBUF, K, DH), f32),
            pltpu.VMEM_SHARED((NROWS_SC, DH), f32),
            pltpu.SemaphoreType.DMA,
            pltpu.SemaphoreType.DMA,
        ],
    )(mlo, mhi, packed2, zeros_rpt)


def _mm_body(x_ref, w_ref, lo_ref, hi_ref):
    m = jnp.dot(x_ref[...], w_ref[...], preferred_element_type=f32)
    lo_ref[...] = m[:, :DH]
    hi_ref[...] = m[:, DH:]


def _mm(x, w):
    return pl.pallas_call(
        _mm_body,
        grid=(GRID,),
        in_specs=[pl.BlockSpec((BLK, D), lambda i: (i, 0)),
                  pl.BlockSpec((D, D), lambda i: (0, 0))],
        out_specs=[pl.BlockSpec((BLK, DH), lambda i: (i, 0)),
                   pl.BlockSpec((BLK, DH), lambda i: (i, 0))],
        out_shape=[jax.ShapeDtypeStruct((N, DH), f32),
                   jax.ShapeDtypeStruct((N, DH), f32)],
    )(x, w)


def _gru_compute(plo_ref, phi_ref, h_ref, w_ref, b_ref):
    agg = jnp.concatenate([plo_ref[...], phi_ref[...]], axis=1)
    h = h_ref[...]
    dot = functools.partial(jnp.dot, preferred_element_type=f32)
    r = jax.nn.sigmoid(dot(agg, w_ref[0]) + dot(h, w_ref[3]) + b_ref[0:1, :])
    z = jax.nn.sigmoid(dot(agg, w_ref[1]) + dot(h, w_ref[4]) + b_ref[1:2, :])
    hn = dot(h, w_ref[5]) + b_ref[3:4, :]
    n = jnp.tanh(dot(agg, w_ref[2]) + b_ref[2:3, :] + r * hn)
    return (1.0 - z) * n + z * h


def _gru_body_m(plo_ref, phi_ref, h_ref, w_ref, b_ref, h_out, mlo_out,
                mhi_out):
    hnew = _gru_compute(plo_ref, phi_ref, h_ref, w_ref, b_ref)
    h_out[...] = hnew
    m = jnp.dot(hnew, w_ref[6], preferred_element_type=f32)
    mlo_out[...] = m[:, :DH]
    mhi_out[...] = m[:, DH:]


def _gru_body_last(plo_ref, phi_ref, h_ref, w_ref, b_ref, h_out):
    h_out[...] = _gru_compute(plo_ref, phi_ref, h_ref, w_ref, b_ref)


def _gru_step(plo, phi, h, ws, b, emit_m):
    nw = ws.shape[0]
    if emit_m:
        out_shape = [jax.ShapeDtypeStruct((N, D), f32),
                     jax.ShapeDtypeStruct((N, DH), f32),
                     jax.ShapeDtypeStruct((N, DH), f32)]
        out_specs = [pl.BlockSpec((BLK, D), lambda i: (i, 0)),
                     pl.BlockSpec((BLK, DH), lambda i: (i, 0)),
                     pl.BlockSpec((BLK, DH), lambda i: (i, 0))]
        body = _gru_body_m
    else:
        out_shape = [jax.ShapeDtypeStruct((N, D), f32)]
        out_specs = [pl.BlockSpec((BLK, D), lambda i: (i, 0))]
        body = _gru_body_last
    return pl.pallas_call(
        body,
        grid=(GRID,),
        in_specs=[pl.BlockSpec((BLK, DH), lambda i: (i, 0)),
                  pl.BlockSpec((BLK, DH), lambda i: (i, 0)),
                  pl.BlockSpec((BLK, D), lambda i: (i, 0)),
                  pl.BlockSpec((nw, D, D), lambda i: (0, 0, 0)),
                  pl.BlockSpec((8, D), lambda i: (0, 0))],
        out_specs=out_specs,
        out_shape=out_shape,
    )(plo, phi, h, ws, b)


def _tail_body(h_ref, batch_ref, l1w_ref, l1b_ref, f1w_ref, f1b_ref,
               f2w_ref, f2b_ref, clsw_ref, clsb_ref, y_ref, acc):
    i = pl.program_id(0)

    @pl.when(i == 0)
    def _():
        acc[...] = jnp.zeros_like(acc)

    out = jax.nn.relu(h_ref[...])
    b = batch_ref[0, 0, :]
    seg = lax.broadcasted_iota(jnp.int32, (G, BLK), 0)
    onehot = jnp.where(seg == b[None, :], 1.0, 0.0).astype(f32)
    acc[...] += jnp.dot(onehot, out, preferred_element_type=f32)

    @pl.when(i == GRID - 1)
    def _():
        dot = functools.partial(jnp.dot, preferred_element_type=f32)
        pooled = acc[...]
        a = jax.nn.relu(dot(pooled, l1w_ref[...]) + l1b_ref[0:1, :])
        a = jax.nn.relu(dot(a, f1w_ref[...]) + f1b_ref[0:1, :])
        a = jax.nn.relu(dot(a, f2w_ref[...]) + f2b_ref[0:1, :])
        logits = dot(a, clsw_ref[...]) + clsb_ref[0:1, :]
        lane = lax.broadcasted_iota(jnp.int32, (G, 8), 1)
        logits = jnp.where(lane < 2, logits, -1e30)
        mx = jnp.max(logits, axis=1, keepdims=True)
        e = jnp.exp(logits - mx)
        y_ref[...] = e / jnp.sum(e, axis=1, keepdims=True)


def _tail(h, batch3, l1w, l1b, f1w, f1b, f2w, f2b, clsw, clsb):
    return pl.pallas_call(
        _tail_body,
        grid=(GRID,),
        in_specs=[pl.BlockSpec((BLK, D), lambda i: (i, 0)),
                  pl.BlockSpec((1, 1, BLK), lambda i: (i, 0, 0)),
                  pl.BlockSpec((D, HID), lambda i: (0, 0)),
                  pl.BlockSpec((1, HID), lambda i: (0, 0)),
                  pl.BlockSpec((HID, D), lambda i: (0, 0)),
                  pl.BlockSpec((1, D), lambda i: (0, 0)),
                  pl.BlockSpec((D, HID), lambda i: (0, 0)),
                  pl.BlockSpec((1, HID), lambda i: (0, 0)),
                  pl.BlockSpec((HID, 8), lambda i: (0, 0)),
                  pl.BlockSpec((1, 8), lambda i: (0, 0))],
        out_specs=pl.BlockSpec((G, 8), lambda i: (0, 0)),
        out_shape=jax.ShapeDtypeStruct((G, 8), f32),
        scratch_shapes=[pltpu.VMEM((G, D), f32)],
    )(h, batch3, l1w, l1b, f1w, f1b, f2w, f2b, clsw, clsb)


def kernel(x, edge_index, batch, ggnn_weight, W_ih, W_hh, b_ih, b_hh,
           l1_w, l1_b, f1_w, f1_b, f2_w, f2_b, cls_w, cls_b):
    h0 = jnp.pad(x, ((0, 0), (0, D - IN))).astype(f32)
    src = jnp.concatenate([edge_index[0],
                           jnp.zeros((E_PAD - E,), jnp.int32)])
    dst = jnp.concatenate([edge_index[1],
                           jnp.full((E_PAD - E,), DUMMY_DST, jnp.int32)])
    packed2 = ((dst << 14) | src).reshape(E_PAD // K, K)
    zeros_rpt = jnp.zeros((RPT, DH), f32)
    batch3 = batch.reshape(GRID, 1, BLK)

    wg = [_pad2(ggnn_weight[i], D, D) for i in range(STEPS)]
    wir = _pad2(W_ih[0:OUT].T, D, D)
    wiz = _pad2(W_ih[OUT:2 * OUT].T, D, D)
    win = _pad2(W_ih[2 * OUT:].T, D, D)
    whr = _pad2(W_hh[0:OUT].T, D, D)
    whz = _pad2(W_hh[OUT:2 * OUT].T, D, D)
    whn = _pad2(W_hh[2 * OUT:].T, D, D)
    br = jnp.pad(b_ih[0:OUT] + b_hh[0:OUT], (0, D - OUT))
    bz = jnp.pad(b_ih[OUT:2 * OUT] + b_hh[OUT:2 * OUT], (0, D - OUT))
    bin_ = jnp.pad(b_ih[2 * OUT:], (0, D - OUT))
    bhn = jnp.pad(b_hh[2 * OUT:], (0, D - OUT))
    bmat = jnp.stack([br, bz, bin_, bhn] + [jnp.zeros((D,), f32)] * 4)

    l1wt = _pad2(l1_w.T, D, HID)
    f1wt = _pad2(f1_w.T, HID, D)
    f2wt = _pad2(f2_w.T, D, HID)
    clswt = _pad2(cls_w.T, HID, 8)
    l1b2 = l1_b.reshape(1, HID)
    f1b2 = _pad2(f1_b.reshape(1, OUT), 1, D)
    f2b2 = f2_b.reshape(1, HID)
    clsb2 = _pad2(cls_b.reshape(1, 2), 1, 8)

    h = h0
    mlo, mhi = _mm(h, wg[0])
    for i in range(STEPS):
        plo, phi = _sc_scatter(mlo, mhi, packed2, zeros_rpt)
        if i < STEPS - 1:
            ws = jnp.stack([wir, wiz, win, whr, whz, whn, wg[i + 1]])
            h, mlo, mhi = _gru_step(plo, phi, h, ws, bmat, True)
        else:
            ws = jnp.stack([wir, wiz, win, whr, whz, whn])
            (h,) = _gru_step(plo, phi, h, ws, bmat, False)

    y8 = _tail(h, batch3, l1wt, l1b2, f1wt, f1b2, f2wt, f2b2, clswt, clsb2)
    return y8[:, :2]
